# baseline (speedup 1.0000x reference)
import functools

import jax
import jax.numpy as jnp
from jax import lax
from jax.experimental import pallas as pl
from jax.experimental.pallas import tpu as pltpu
from jax.experimental.pallas import tpu_sc as plsc

CHUNK = 128
TCH = 2


@functools.lru_cache(maxsize=None)
def _build(V, D, B, n_rows):
    info = plsc.get_sparse_core_info()
    nc, ns, L = info.num_cores, info.num_subcores, info.num_lanes
    nw = nc * ns
    bpw = B // nw
    nch = bpw // CHUNK
    NT = -(-n_rows // 128)
    N2 = NT * 128
    TPW = -(-NT // nw)
    nstep = -(-TPW // TCH)
    mesh = plsc.VectorSubcoreMesh(core_axis_name="c", subcore_axis_name="s")
    params = pltpu.CompilerParams(needs_layout_passes=False)
    params_lin = pltpu.CompilerParams(
        use_tc_tiling_on_sc=False, needs_layout_passes=False
    )

    @functools.partial(
        pl.kernel,
        mesh=mesh,
        out_type=jax.ShapeDtypeStruct((N2 * D,), jnp.float32),
        scratch_types=[
            pltpu.VMEM((2, D, TCH * 128), jnp.float32),
            pltpu.VMEM((2 * TCH * 128 * D,), jnp.float32),
            pltpu.SemaphoreType.DMA,
            pltpu.SemaphoreType.DMA,
        ],
        compiler_params=params,
    )
    def detile(wt, o1, sbuf, rbuf, isem, osem):
        wid = lax.axis_index("s") * nc + lax.axis_index("c")
        t0 = jnp.minimum(wid * TPW, NT - TPW)
        lanes = lax.iota(jnp.int32, L)
        RC = TCH * 128

        def col0(step):
            return (t0 + jnp.minimum(step * TCH, TPW - TCH)) * 128

        def in_copies(step, buf):
            c0 = col0(step)
            return [
                pltpu.make_async_copy(
                    wt.at[pl.ds(8 * q, 8), pl.ds(c0, RC)],
                    sbuf.at[buf, pl.ds(8 * q, 8)],
                    isem,
                )
                for q in range(D // 8)
            ]

        def out_copy(step, buf):
            return pltpu.make_async_copy(
                rbuf.at[pl.ds(buf * RC * D, RC * D)],
                o1.at[pl.ds(col0(step) * D, RC * D)],
                osem,
            )

        for c in in_copies(0, 0):
            c.start()

        def step_fn(step, buf):
            for c in in_copies(step, buf):
                c.wait()

            @pl.when(step + 1 < nstep)
            def _():
                for c in in_copies(step + 1, 1 - buf):
                    c.start()

            @pl.when(step >= 2)
            def _():
                out_copy(step - 2, buf).wait()

            def row_group(g, carry):
                ivec = lanes + g * L
                obase = ivec * D
                for cg in range(D // L):
                    cvec = lanes + cg * L
                    for u in range(L):
                        vals = plsc.load_gather(
                            sbuf.at[buf],
                            [cvec, jnp.full((L,), 0, jnp.int32) + (g * L + u)],
                        )
                        plsc.store_scatter(
                            rbuf,
                            [
                                jnp.full(
                                    (L,),
                                    buf * RC * D + (g * L + u) * D + cg * L,
                                    jnp.int32,
                                )
                                + lanes
                            ],
                            vals,
                        )
                return carry

            lax.fori_loop(0, RC // L, row_group, jnp.int32(0))
            out_copy(step, buf).start()

        def pair(jj, carry):
            for phase in range(2):
                step_fn(jj * 2 + phase, phase)
            return carry

        lax.fori_loop(0, nstep // 2, pair, jnp.int32(0))
        if nstep % 2:
            step_fn(nstep - 1, 0)
        out_copy(nstep - 2, nstep % 2).wait()
        out_copy(nstep - 1, (nstep - 1) % 2).wait()

    @functools.partial(
        pl.kernel,
        mesh=mesh,
        out_type=jax.ShapeDtypeStruct((D, B), jnp.float32),
        scratch_types=[
            pltpu.VMEM((bpw,), jnp.int32),
            pltpu.VMEM((2, CHUNK, D), jnp.float32),
            pltpu.VMEM((D, bpw), jnp.float32),
            pltpu.SemaphoreType.DMA,
            pltpu.SemaphoreType.DMA,
        ],
        compiler_params=params_lin,
    )
    def gather(table, idx_hbm, out_hbm, idx_v, rbuf, tbuf, gsem, osem):
        wid = lax.axis_index("s") * nc + lax.axis_index("c")
        base = wid * bpw
        pltpu.sync_copy(idx_hbm.at[pl.ds(base, bpw)], idx_v)
        lanes = lax.iota(jnp.int32, L)

        def g_copy(j, buf):
            return pltpu.make_async_copy(
                table.at[idx_v.at[pl.ds(j * CHUNK, CHUNK)]], rbuf.at[buf], gsem
            )

        g_copy(0, 0).start()
        for j in range(nch):
            buf = j % 2
            g_copy(j, buf).wait()
            if j + 1 < nch:
                g_copy(j + 1, 1 - buf).start()

            def xpose(g, carry):
                jvec = lanes + g * L
                for c in range(D):
                    vals = plsc.load_gather(
                        rbuf.at[buf], [jvec, jnp.full((L,), c, jnp.int32)]
                    )
                    tbuf[c, pl.ds(j * CHUNK + g * L, L)] = vals
                return carry

            lax.fori_loop(0, CHUNK // L, xpose, jnp.int32(0))
        pltpu.sync_copy(tbuf, out_hbm.at[:, pl.ds(base, bpw)])

    JB = B // (nw // (D // 8))

    @functools.partial(
        pl.kernel,
        mesh=mesh,
        out_type=jax.ShapeDtypeStruct((D, B), jnp.float32),
        scratch_types=[
            pltpu.VMEM((8 * JB,), jnp.float32),
            pltpu.VMEM((2, 8, 128), jnp.float32),
            pltpu.SemaphoreType.DMA,
            pltpu.SemaphoreType.DMA,
        ],
        compiler_params=params,
    )
    def retile(in1d, oc, sbufc, tbufc, isem, osem):
        wid = lax.axis_index("s") * nc + lax.axis_index("c")
        nq = D // 8
        q = wid % nq
        j0 = (wid // nq) * JB
        for r in range(8):
            pltpu.async_copy(
                in1d.at[pl.ds((8 * q + r) * B + j0, JB)],
                sbufc.at[pl.ds(r * JB, JB)],
                isem,
            )
        for r in range(8):
            pltpu.make_async_copy(
                in1d.at[pl.ds((8 * q + r) * B + j0, JB)],
                sbufc.at[pl.ds(r * JB, JB)],
                isem,
            ).wait()

        def tile_fn(jt, carry):
            buf = jt % 2

            @pl.when(jt >= 2)
            def _():
                pltpu.make_async_copy(
                    tbufc.at[buf],
                    oc.at[pl.ds(8 * q, 8), pl.ds(j0, 128)],
                    osem,
                ).wait()

            for r in range(8):
                for k in range(128 // L):
                    tbufc[buf, r, pl.ds(k * L, L)] = sbufc[
                        pl.ds(r * JB + jt * 128 + k * L, L)
                    ]
            pltpu.async_copy(
                tbufc.at[buf],
                oc.at[pl.ds(8 * q, 8), pl.ds(j0 + jt * 128, 128)],
                osem,
            )
            return carry

        lax.fori_loop(0, JB // 128, tile_fn, jnp.int32(0))
        for _ in range(2):
            pltpu.make_async_copy(
                tbufc.at[0], oc.at[pl.ds(8 * q, 8), pl.ds(j0, 128)], osem
            ).wait()

    def run(weight, subset):
        wt = weight.T
        flat = detile(wt)
        table = flat.reshape(N2, D)
        out_t = gather(table, subset.astype(jnp.int32))
        out_c = retile(out_t.reshape(D * B))
        return out_c.T

    return run


def kernel(weight, subset):
    return _build(weight.shape[0], weight.shape[1], subset.shape[0], 500000)(
        weight, subset
    )
